# baseline (speedup 1.0000x reference)
import functools

import jax
import jax.numpy as jnp
from jax import lax
from jax.experimental import pallas as pl
from jax.experimental.pallas import tpu as pltpu
from jax.experimental.pallas import tpu_sc as plsc

BATCH = 4096
HIST_LEN = 20
VOCAB = 1000

NUM_CORES = 2
NUM_SUBCORES = 16
NUM_WORKERS = NUM_CORES * NUM_SUBCORES
COLS_PER_WORKER = BATCH // NUM_WORKERS
LANES = 16
GROUPS = COLS_PER_WORKER // LANES
VST_ROWS = 344
DMA_ROWS = VOCAB - VST_ROWS


def _sc_body(xt_hbm, xtail_hbm, zeros_hbm, out_hbm, idx_v, hist_v, sem, zsem):
    c = lax.axis_index("c")
    s = lax.axis_index("s")
    wid = s * NUM_CORES + c
    col_base = wid * COLS_PER_WORKER

    lanes = lax.iota(jnp.int32, LANES)
    ones = jnp.ones((LANES,), jnp.float32)
    zeros = jnp.zeros((LANES,), jnp.float32)

    zero_copy = pltpu.async_copy(
        zeros_hbm,
        hist_v.at[pl.ds(VST_ROWS, DMA_ROWS), :],
        zsem,
    )
    stage_copy = pltpu.async_copy(
        xt_hbm.at[pl.ds(0, 16), pl.ds(col_base, COLS_PER_WORKER)],
        idx_v,
        sem,
    )

    def clear_step(i, _):
        v8 = i * 8
        for dv in range(8):
            for ch in range(GROUPS):
                hist_v[v8 + dv, pl.ds(ch * LANES, LANES)] = zeros
        return 0

    lax.fori_loop(0, VST_ROWS // 8, clear_step, 0)
    zero_copy.wait()
    stage_copy.wait()

    def scatter_pass(l_lo, l_hi, stage_off):
        for g in range(GROUPS):
            cols = lanes + g * LANES
            for l in range(l_lo, l_hi):
                row = jnp.full((LANES,), l - stage_off, jnp.int32)
                v = plsc.load_gather(idx_v, [row, cols])
                plsc.addupdate_scatter(hist_v, [v, cols], ones)

    scatter_pass(0, 16, 0)
    pltpu.sync_copy(
        xtail_hbm.at[:, pl.ds(col_base, COLS_PER_WORKER)],
        idx_v.at[pl.ds(0, 8), :],
    )
    scatter_pass(16, HIST_LEN, 12)

    pltpu.async_copy(
        hist_v,
        out_hbm.at[:, pl.ds(col_base, COLS_PER_WORKER)],
        sem,
    ).wait()


def _make_sc_kernel():
    mesh = plsc.VectorSubcoreMesh(core_axis_name="c", subcore_axis_name="s")
    return functools.partial(
        pl.kernel,
        mesh=mesh,
        out_type=jax.ShapeDtypeStruct((VOCAB, BATCH), jnp.float32),
        scratch_types=[
            pltpu.VMEM((16, COLS_PER_WORKER), jnp.int32),
            pltpu.VMEM((VOCAB, COLS_PER_WORKER), jnp.float32),
            pltpu.SemaphoreType.DMA,
            pltpu.SemaphoreType.DMA,
        ],
        compiler_params=pltpu.CompilerParams(
            needs_layout_passes=False, use_tc_tiling_on_sc=True
        ),
    )(_sc_body)


_sc_kernel = _make_sc_kernel()


@jax.jit
def kernel(x):
    xt = x.T
    xtail = x[:, HIST_LEN - 8 :].T
    zeros_hbm = jnp.zeros((DMA_ROWS, COLS_PER_WORKER), jnp.float32)
    return _sc_kernel(xt, xtail, zeros_hbm).T


# speedup vs baseline: 1.0570x; 1.0570x over previous
import functools

import jax
import jax.numpy as jnp
from jax import lax
from jax.experimental import pallas as pl
from jax.experimental.pallas import tpu as pltpu
from jax.experimental.pallas import tpu_sc as plsc

BATCH = 4096
HIST_LEN = 20
VOCAB = 1000

NUM_CORES = 2
NUM_SUBCORES = 16
NUM_WORKERS = NUM_CORES * NUM_SUBCORES
COLS_PER_WORKER = BATCH // NUM_WORKERS
LANES = 16
GROUPS = COLS_PER_WORKER // LANES
VST_ROWS = 600
DMA_ROWS = VOCAB - VST_ROWS


def _sc_body(xt_hbm, xtail_hbm, zeros_hbm, out_hbm, idx_v, hist_v, sem, zsem):
    c = lax.axis_index("c")
    s = lax.axis_index("s")
    wid = s * NUM_CORES + c
    col_base = wid * COLS_PER_WORKER

    lanes = lax.iota(jnp.int32, LANES)
    ones = jnp.ones((LANES,), jnp.float32)
    zeros = jnp.zeros((LANES,), jnp.float32)

    zero_copy = pltpu.async_copy(
        zeros_hbm,
        hist_v.at[pl.ds(VST_ROWS, DMA_ROWS), :],
        zsem,
    )
    stage_copy = pltpu.async_copy(
        xt_hbm.at[pl.ds(0, 16), pl.ds(col_base, COLS_PER_WORKER)],
        idx_v,
        sem,
    )

    def clear_step(i, _):
        v8 = i * 8
        for dv in range(8):
            for ch in range(GROUPS):
                hist_v[v8 + dv, pl.ds(ch * LANES, LANES)] = zeros
        return 0

    lax.fori_loop(0, VST_ROWS // 8, clear_step, 0)
    zero_copy.wait()
    stage_copy.wait()

    def scatter_pass(l_lo, l_hi, stage_off):
        for g in range(GROUPS):
            cols = lanes + g * LANES
            for l in range(l_lo, l_hi):
                row = jnp.full((LANES,), l - stage_off, jnp.int32)
                v = plsc.load_gather(idx_v, [row, cols])
                plsc.addupdate_scatter(hist_v, [v, cols], ones)

    scatter_pass(0, 16, 0)
    pltpu.sync_copy(
        xtail_hbm.at[:, pl.ds(col_base, COLS_PER_WORKER)],
        idx_v.at[pl.ds(0, 8), :],
    )
    scatter_pass(16, HIST_LEN, 12)

    pltpu.async_copy(
        hist_v,
        out_hbm.at[:, pl.ds(col_base, COLS_PER_WORKER)],
        sem,
    ).wait()


def _make_sc_kernel():
    mesh = plsc.VectorSubcoreMesh(core_axis_name="c", subcore_axis_name="s")
    return functools.partial(
        pl.kernel,
        mesh=mesh,
        out_type=jax.ShapeDtypeStruct((VOCAB, BATCH), jnp.float32),
        scratch_types=[
            pltpu.VMEM((16, COLS_PER_WORKER), jnp.int32),
            pltpu.VMEM((VOCAB, COLS_PER_WORKER), jnp.float32),
            pltpu.SemaphoreType.DMA,
            pltpu.SemaphoreType.DMA,
        ],
        compiler_params=pltpu.CompilerParams(
            needs_layout_passes=False, use_tc_tiling_on_sc=True
        ),
    )(_sc_body)


_sc_kernel = _make_sc_kernel()


@jax.jit
def kernel(x):
    xt = x.T
    xtail = x[:, HIST_LEN - 8 :].T
    zeros_hbm = jnp.zeros((DMA_ROWS, COLS_PER_WORKER), jnp.float32)
    return _sc_kernel(xt, xtail, zeros_hbm).T


# speedup vs baseline: 1.1783x; 1.1148x over previous
import functools

import jax
import jax.numpy as jnp
from jax import lax
from jax.experimental import pallas as pl
from jax.experimental.pallas import tpu as pltpu
from jax.experimental.pallas import tpu_sc as plsc

BATCH = 4096
HIST_LEN = 20
VOCAB = 1000

NUM_CORES = 2
NUM_SUBCORES = 16
NUM_WORKERS = NUM_CORES * NUM_SUBCORES
COLS_PER_WORKER = BATCH // NUM_WORKERS
LANES = 16
GROUPS = COLS_PER_WORKER // LANES
VST_ROWS = 720
DMA_ROWS = VOCAB - VST_ROWS


def _sc_body(xt_hbm, xtail_hbm, zeros_hbm, out_hbm, idx_v, hist_v, sem, zsem):
    c = lax.axis_index("c")
    s = lax.axis_index("s")
    wid = s * NUM_CORES + c
    col_base = wid * COLS_PER_WORKER

    lanes = lax.iota(jnp.int32, LANES)
    ones = jnp.ones((LANES,), jnp.float32)
    zeros = jnp.zeros((LANES,), jnp.float32)

    zero_copy = pltpu.async_copy(
        zeros_hbm.at[s % 4],
        hist_v.at[pl.ds(VST_ROWS, DMA_ROWS), :],
        zsem,
    )
    stage_copy = pltpu.async_copy(
        xt_hbm.at[pl.ds(0, 16), pl.ds(col_base, COLS_PER_WORKER)],
        idx_v,
        sem,
    )

    def clear_step(i, _):
        v8 = i * 8
        for dv in range(8):
            for ch in range(GROUPS):
                hist_v[v8 + dv, pl.ds(ch * LANES, LANES)] = zeros
        return 0

    lax.fori_loop(0, VST_ROWS // 8, clear_step, 0)
    zero_copy.wait()
    stage_copy.wait()

    def scatter_pass(l_lo, l_hi, stage_off):
        for g in range(GROUPS):
            cols = lanes + g * LANES
            for l in range(l_lo, l_hi):
                row = jnp.full((LANES,), l - stage_off, jnp.int32)
                v = plsc.load_gather(idx_v, [row, cols])
                plsc.addupdate_scatter(hist_v, [v, cols], ones)

    scatter_pass(0, 16, 0)
    pltpu.sync_copy(
        xtail_hbm.at[:, pl.ds(col_base, COLS_PER_WORKER)],
        idx_v.at[pl.ds(0, 8), :],
    )
    scatter_pass(16, HIST_LEN, 12)

    pltpu.async_copy(
        hist_v,
        out_hbm.at[:, pl.ds(col_base, COLS_PER_WORKER)],
        sem,
    ).wait()


def _make_sc_kernel():
    mesh = plsc.VectorSubcoreMesh(core_axis_name="c", subcore_axis_name="s")
    return functools.partial(
        pl.kernel,
        mesh=mesh,
        out_type=jax.ShapeDtypeStruct((VOCAB, BATCH), jnp.float32),
        scratch_types=[
            pltpu.VMEM((16, COLS_PER_WORKER), jnp.int32),
            pltpu.VMEM((VOCAB, COLS_PER_WORKER), jnp.float32),
            pltpu.SemaphoreType.DMA,
            pltpu.SemaphoreType.DMA,
        ],
        compiler_params=pltpu.CompilerParams(
            needs_layout_passes=False, use_tc_tiling_on_sc=True
        ),
    )(_sc_body)


_sc_kernel = _make_sc_kernel()


@jax.jit
def kernel(x):
    xt = x.T
    xtail = x[:, HIST_LEN - 8 :].T
    zeros_hbm = jnp.zeros((4, DMA_ROWS, COLS_PER_WORKER), jnp.float32)
    return _sc_kernel(xt, xtail, zeros_hbm).T
